# Initial kernel scaffold; baseline (speedup 1.0000x reference)
import jax
import jax.numpy as jnp
from jax.experimental import pallas as pl


def kernel(x, edge_index, edge_attr, lstm_Wih, lstm_Whh, lstm_bih, lstm_bhh, Wsrc, att_src, att_dst, Wedge, att_edge, gat_bias, ln_g, ln_b, fc1_w, fc1_b, fc2_w, fc2_b):
    raise NotImplementedError("write your pallas kernel here")


# speedup vs baseline: 1.0085x; 1.0085x over previous
import functools

import jax
import jax.numpy as jnp
from jax.experimental import pallas as pl
from jax.experimental.pallas import tpu as pltpu

N = 50000
E = 800000
T = 24
C_IN = 2
H = 64
ED = 2
L = 5
C_OUT = 2

LSTM_BN = 2000


def _lstm_block_kernel(x_ref, wih_ref, whh_ref, b_ref, out_ref, *, bn):
    h = jnp.zeros((bn, H), dtype=jnp.float32)
    c = jnp.zeros((bn, H), dtype=jnp.float32)
    wih = wih_ref[...]
    whh = whh_ref[...]
    b = b_ref[...]
    for t in range(T):
        xt = x_ref[:, t, :]
        gates = (
            jax.lax.dot_general(xt, wih, (((1,), (0,)), ((), ())),
                                preferred_element_type=jnp.float32)
            + jax.lax.dot_general(h, whh, (((1,), (0,)), ((), ())),
                                  preferred_element_type=jnp.float32)
            + b
        )
        i = jax.nn.sigmoid(gates[:, 0:H])
        f = jax.nn.sigmoid(gates[:, H:2 * H])
        g = jnp.tanh(gates[:, 2 * H:3 * H])
        o = jax.nn.sigmoid(gates[:, 3 * H:4 * H])
        c = f * c + i * g
        h = o * jnp.tanh(c)
    out_ref[...] = h


def _lstm_pallas(x, Wih, Whh, bih, bhh):
    n = x.shape[0]
    bn = LSTM_BN
    nblocks = n // bn
    wih_t = Wih.T
    whh_t = Whh.T
    b = (bih + bhh).reshape(1, 4 * H)
    return pl.pallas_call(
        functools.partial(_lstm_block_kernel, bn=bn),
        grid=(nblocks,),
        in_specs=[
            pl.BlockSpec((bn, T, C_IN), lambda i: (i, 0, 0)),
            pl.BlockSpec((C_IN, 4 * H), lambda i: (0, 0)),
            pl.BlockSpec((H, 4 * H), lambda i: (0, 0)),
            pl.BlockSpec((1, 4 * H), lambda i: (0, 0)),
        ],
        out_specs=pl.BlockSpec((bn, H), lambda i: (i, 0)),
        out_shape=jax.ShapeDtypeStruct((n, H), jnp.float32),
    )(x, wih_t, whh_t, b)


def _gat_layer(x, src, dst, edge_attr, W, a_src, a_dst, We, a_e, b):
    xs = x @ W
    asrc = jnp.sum(xs * a_src, axis=-1)
    adst = jnp.sum(xs * a_dst, axis=-1)
    e = edge_attr @ We
    ae = jnp.sum(e * a_e, axis=-1)
    alpha = jax.nn.leaky_relu(asrc[src] + adst[dst] + ae, negative_slope=0.2)
    amax = jax.ops.segment_max(alpha, dst, num_segments=N)
    amax = jnp.where(jnp.isfinite(amax), amax, 0.0)
    ex = jnp.exp(alpha - amax[dst])
    denom = jax.ops.segment_sum(ex, dst, num_segments=N)
    coef = ex / (denom[dst] + 1e-16)
    out = jax.ops.segment_sum(coef[:, None] * xs[src], dst, num_segments=N)
    return out + b


def _layer_norm(x, g, b):
    mu = jnp.mean(x, axis=-1, keepdims=True)
    var = jnp.var(x, axis=-1, keepdims=True)
    return (x - mu) / jnp.sqrt(var + 1e-5) * g + b


def kernel(x, edge_index, edge_attr, lstm_Wih, lstm_Whh, lstm_bih, lstm_bhh,
           Wsrc, att_src, att_dst, Wedge, att_edge, gat_bias, ln_g, ln_b,
           fc1_w, fc1_b, fc2_w, fc2_b):
    src = edge_index[0]
    dst = edge_index[1]
    h = _lstm_pallas(x, lstm_Wih, lstm_Whh, lstm_bih, lstm_bhh)
    xs_list = []
    cur = h
    for i in range(L):
        cur = _gat_layer(cur, src, dst, edge_attr, Wsrc[i], att_src[i],
                         att_dst[i], Wedge[i], att_edge[i], gat_bias[i])
        cur = _layer_norm(cur, ln_g[i], ln_b[i])
        cur = jax.nn.relu(cur)
        xs_list.append(cur)
    x_jk = jnp.concatenate(xs_list, axis=-1)
    h1 = jax.nn.relu(x_jk @ fc1_w + fc1_b)
    out = h1 @ fc2_w + fc2_b
    return out
